# Initial kernel scaffold; baseline (speedup 1.0000x reference)
import jax
import jax.numpy as jnp
from jax.experimental import pallas as pl


def kernel(x):
    raise NotImplementedError("write your pallas kernel here")


# speedup vs baseline: 3.6046x; 3.6046x over previous
import functools

import jax
import jax.numpy as jnp
from jax import lax
from jax.experimental import pallas as pl
from jax.experimental.pallas import tpu as pltpu
from jax.experimental.pallas import tpu_sc as plsc

ROWS = 128
COLS = 32768
LANES = 16
CHUNKS = COLS // LANES
UNROLL = 8

_info = plsc.get_sparse_core_info()
_NC, _NS = _info.num_cores, _info.num_subcores
NW = _NC * _NS
ROWS_PER_W = ROWS // NW

_mesh = plsc.VectorSubcoreMesh(core_axis_name="c", subcore_axis_name="s")


@functools.partial(
    pl.kernel,
    mesh=_mesh,
    out_type=jax.ShapeDtypeStruct((ROWS, COLS), jnp.float32),
    scratch_types=[pltpu.VMEM((COLS,), jnp.float32)],
    compiler_params=pltpu.CompilerParams(needs_layout_passes=False),
)
def _cumsum_sc(x_hbm, out_hbm, buf):
    wid = lax.axis_index("s") * _NC + lax.axis_index("c")

    def do_row(row):
        pltpu.sync_copy(x_hbm.at[row], buf)

        def body(i, carry):
            base = i * (LANES * UNROLL)
            for u in range(UNROLL):
                off = base + u * LANES
                v = buf[pl.ds(off, LANES)]
                s = plsc.cumsum(v)
                buf[pl.ds(off, LANES)] = s + carry
                carry = carry + jnp.sum(v)
            return carry

        lax.fori_loop(0, CHUNKS // UNROLL, body, jnp.zeros((LANES,), jnp.float32))
        pltpu.sync_copy(buf, out_hbm.at[row])

    for k in range(ROWS_PER_W):
        do_row(wid * ROWS_PER_W + k)


def kernel(x):
    return _cumsum_sc(x)


# speedup vs baseline: 4.4393x; 1.2316x over previous
import functools

import jax
import jax.numpy as jnp
from jax import lax
from jax.experimental import pallas as pl
from jax.experimental.pallas import tpu as pltpu
from jax.experimental.pallas import tpu_sc as plsc

ROWS = 128
COLS = 32768
LANES = 16
CHUNKS = COLS // LANES
UNROLL = 8

_info = plsc.get_sparse_core_info()
_NC, _NS = _info.num_cores, _info.num_subcores
NW = _NC * _NS
ROWS_PER_W = ROWS // NW

_mesh = plsc.VectorSubcoreMesh(core_axis_name="c", subcore_axis_name="s")


NBUF = 3


@functools.partial(
    pl.kernel,
    mesh=_mesh,
    out_type=jax.ShapeDtypeStruct((ROWS, COLS), jnp.float32),
    scratch_types=(
        [pltpu.VMEM((COLS,), jnp.float32)] * NBUF
        + [pltpu.SemaphoreType.DMA] * (2 * NBUF)
    ),
    compiler_params=pltpu.CompilerParams(needs_layout_passes=False),
)
def _cumsum_sc(x_hbm, out_hbm, *scratch):
    bufs = scratch[:NBUF]
    isems = scratch[NBUF : 2 * NBUF]
    osems = scratch[2 * NBUF :]
    wid = lax.axis_index("s") * _NC + lax.axis_index("c")
    rows = [wid * ROWS_PER_W + k for k in range(ROWS_PER_W)]

    def scan_row(buf):
        def body(i, carry):
            base = i * (LANES * UNROLL)
            for u in range(UNROLL):
                off = base + u * LANES
                v = buf[pl.ds(off, LANES)]
                s = plsc.cumsum(v)
                buf[pl.ds(off, LANES)] = s + carry
                carry = carry + jnp.sum(v)
            return carry

        lax.fori_loop(0, CHUNKS // UNROLL, body, jnp.zeros((LANES,), jnp.float32))

    in_h, out_h = {}, {}
    in_h[0] = pltpu.async_copy(x_hbm.at[rows[0]], bufs[0], isems[0])
    for k in range(ROWS_PER_W):
        b = k % NBUF
        if k + 1 < ROWS_PER_W:
            nb = (k + 1) % NBUF
            if k + 1 >= NBUF:
                out_h[k + 1 - NBUF].wait()
            in_h[k + 1] = pltpu.async_copy(x_hbm.at[rows[k + 1]], bufs[nb], isems[nb])
        in_h[k].wait()
        scan_row(bufs[b])
        out_h[k] = pltpu.async_copy(bufs[b], out_hbm.at[rows[k]], osems[b])
    for k in range(max(0, ROWS_PER_W - NBUF), ROWS_PER_W):
        out_h[k].wait()


def kernel(x):
    return _cumsum_sc(x)


# speedup vs baseline: 4.4631x; 1.0054x over previous
import functools

import jax
import jax.numpy as jnp
from jax import lax
from jax.experimental import pallas as pl
from jax.experimental.pallas import tpu as pltpu
from jax.experimental.pallas import tpu_sc as plsc

ROWS = 128
COLS = 32768
LANES = 16
CHUNKS = COLS // LANES
UNROLL = 8

_info = plsc.get_sparse_core_info()
_NC, _NS = _info.num_cores, _info.num_subcores
NW = _NC * _NS
ROWS_PER_W = ROWS // NW

_mesh = plsc.VectorSubcoreMesh(core_axis_name="c", subcore_axis_name="s")


TILE = 8192
NT = COLS // TILE
CPT = TILE // LANES


@functools.partial(
    pl.kernel,
    mesh=_mesh,
    out_type=jax.ShapeDtypeStruct((ROWS, COLS), jnp.float32),
    scratch_types=(
        [pltpu.VMEM((ROWS_PER_W, TILE), jnp.float32)] * 2
        + [pltpu.SemaphoreType.DMA] * 4
    ),
    compiler_params=pltpu.CompilerParams(needs_layout_passes=False),
)
def _cumsum_sc(x_hbm, out_hbm, buf_a, buf_b, si0, si1, so0, so1):
    bufs = (buf_a, buf_b)
    isems = (si0, si1)
    osems = (so0, so1)
    wid = lax.axis_index("s") * _NC + lax.axis_index("c")
    rows = [wid * ROWS_PER_W + k for k in range(ROWS_PER_W)]

    def start_in(t):
        p = t % 2
        return [
            pltpu.async_copy(
                x_hbm.at[rows[r], pl.ds(t * TILE, TILE)], bufs[p].at[r], isems[p]
            )
            for r in range(ROWS_PER_W)
        ]

    def scan_tile(p, carries):
        def body(i, carries):
            c = list(carries)
            base = i * (LANES * UNROLL)
            for u in range(UNROLL):
                off = base + u * LANES
                for r in range(ROWS_PER_W):
                    v = bufs[p][r, pl.ds(off, LANES)]
                    s = plsc.cumsum(v)
                    bufs[p][r, pl.ds(off, LANES)] = s + c[r]
                    c[r] = c[r] + jnp.sum(v)
            return tuple(c)

        return lax.fori_loop(0, CPT // UNROLL, body, carries)

    carries = tuple(jnp.zeros((LANES,), jnp.float32) for _ in range(ROWS_PER_W))
    in_h, out_h = {}, {}
    in_h[0] = start_in(0)
    for t in range(NT):
        p = t % 2
        if t + 1 < NT:
            if t - 1 >= 0:
                for h in out_h[t - 1]:
                    h.wait()
            in_h[t + 1] = start_in(t + 1)
        for h in in_h[t]:
            h.wait()
        carries = scan_tile(p, carries)
        out_h[t] = [
            pltpu.async_copy(
                bufs[p].at[r], out_hbm.at[rows[r], pl.ds(t * TILE, TILE)], osems[p]
            )
            for r in range(ROWS_PER_W)
        ]
    for t in range(max(0, NT - 2), NT):
        for h in out_h[t]:
            h.wait()


def kernel(x):
    return _cumsum_sc(x)


# speedup vs baseline: 4.4764x; 1.0030x over previous
import functools

import jax
import jax.numpy as jnp
from jax import lax
from jax.experimental import pallas as pl
from jax.experimental.pallas import tpu as pltpu
from jax.experimental.pallas import tpu_sc as plsc

ROWS = 128
COLS = 32768
LANES = 16
CHUNKS = COLS // LANES
UNROLL = 8

_info = plsc.get_sparse_core_info()
_NC, _NS = _info.num_cores, _info.num_subcores
NW = _NC * _NS
ROWS_PER_W = ROWS // NW

_mesh = plsc.VectorSubcoreMesh(core_axis_name="c", subcore_axis_name="s")


TILE = 8192
NT = COLS // TILE
CPT = TILE // LANES


@functools.partial(
    pl.kernel,
    mesh=_mesh,
    out_type=jax.ShapeDtypeStruct((ROWS, COLS), jnp.float32),
    scratch_types=(
        [pltpu.VMEM((ROWS_PER_W, TILE), jnp.float32)] * 2
        + [pltpu.SemaphoreType.DMA] * 4
    ),
    compiler_params=pltpu.CompilerParams(needs_layout_passes=False),
)
def _cumsum_sc(x_hbm, out_hbm, buf_a, buf_b, si0, si1, so0, so1):
    bufs = (buf_a, buf_b)
    isems = (si0, si1)
    osems = (so0, so1)
    wid = lax.axis_index("s") * _NC + lax.axis_index("c")
    rows = [wid * ROWS_PER_W + k for k in range(ROWS_PER_W)]

    def start_in(t):
        p = t % 2
        return [
            pltpu.async_copy(
                x_hbm.at[rows[r], pl.ds(t * TILE, TILE)], bufs[p].at[r], isems[p]
            )
            for r in range(ROWS_PER_W)
        ]

    def scan_tile(p, carries):
        @plsc.parallel_loop(0, CPT, carry=carries, unroll=UNROLL)
        def final(i, c):
            off = i * LANES
            c = list(c)
            for r in range(ROWS_PER_W):
                v = bufs[p][r, pl.ds(off, LANES)]
                s = plsc.cumsum(v)
                bufs[p][r, pl.ds(off, LANES)] = s + c[r]
                c[r] = c[r] + jnp.sum(v)
            return tuple(c)

        return final

    carries = tuple(jnp.zeros((LANES,), jnp.float32) for _ in range(ROWS_PER_W))
    in_h, out_h = {}, {}
    in_h[0] = start_in(0)
    for t in range(NT):
        p = t % 2
        if t + 1 < NT:
            if t - 1 >= 0:
                for h in out_h[t - 1]:
                    h.wait()
            in_h[t + 1] = start_in(t + 1)
        for h in in_h[t]:
            h.wait()
        carries = scan_tile(p, carries)
        out_h[t] = [
            pltpu.async_copy(
                bufs[p].at[r], out_hbm.at[rows[r], pl.ds(t * TILE, TILE)], osems[p]
            )
            for r in range(ROWS_PER_W)
        ]
    for t in range(max(0, NT - 2), NT):
        for h in out_h[t]:
            h.wait()


def kernel(x):
    return _cumsum_sc(x)
